# baseline (speedup 1.0000x reference)
import functools

import jax
import jax.numpy as jnp
from jax import lax
from jax.experimental import pallas as pl
from jax.experimental.pallas import tpu as pltpu
from jax.experimental.pallas import tpu_sc as plsc

_NU = 5000
_NI = 5000
_N = 10000
_E = 160000
_NNZ = 320000
_D = 256

_NC = 2
_NS = 16

_S = 32
_W = 8
_XROWS = (_N + 8) * _S
_EPAD = 163840
_AVROWS = 10240
_NNZP = 327680
_PPT = _NNZP // _NS
_CH = 1024
_NCHUNK = _PPT // _CH
_IDXR = _CH // 128
_ECHT = _E // _NS
_ECH = 2000
_NECH = _ECHT // _ECH
_EZT = _EPAD // _NS
_ZR = 320
_AVZT = _AVROWS // _NS
_OV0 = 624
_OV1 = 640


def _sca_body(compute_w, *refs):
    if compute_w:
        (xsl, vidx32, eidx, dege, xe_out, w_out,
         acc_e, cnt_sh, rows, zrows, zbuf, ones,
         vi2, ei2, va2, wcv, dcv, sem1, sem2) = refs
    else:
        (xsl, vidx32, eidx, xe_out,
         acc_e, rows, zrows, zbuf, ones,
         vi2, ei2, va2, sem1, sem2) = refs

    c = lax.axis_index("c")
    t = lax.axis_index("s")

    zf = jnp.zeros((16,), jnp.float32)
    of = jnp.ones((16,), jnp.float32)

    def _fill(i, _):
        zbuf[pl.ds(i * 16, 16)] = zf
        return 0
    lax.fori_loop(0, 40, _fill, 0)
    for i in range(8):
        ones[pl.ds(i * 16, 16)] = of
    for i in range(_ZR // 16):
        pltpu.sync_copy(xsl.at[pl.ds(_N * _S, 16), :],
                        zrows.at[pl.ds(i * 16, 16), :])

    if compute_w:
        def _zw(m, _):
            pltpu.sync_copy(zbuf, cnt_sh.at[pl.ds(t * _EZT + m * 640, 640)])
            return 0
        lax.fori_loop(0, _EZT // 640, _zw, 0)
        plsc.subcore_barrier()

        def _cnt(k, _):
            r0 = t * (_PPT // 128) + k * _IDXR
            pltpu.sync_copy(eidx.at[pl.ds(r0, _IDXR), :], ei2)
            hs = [pltpu.async_copy(ones.at[pl.ds(0, 128)],
                                   cnt_sh.at[ei2.at[j]], sem2, add=True)
                  for j in range(_IDXR)]
            for h in hs:
                h.wait()
            return 0
        lax.fori_loop(0, _NCHUNK, _cnt, 0)
        plsc.subcore_barrier()

        def _wc(k, _):
            b = t * _ECHT + k * _ECH
            pltpu.sync_copy(cnt_sh.at[pl.ds(b, _ECH)], wcv.at[pl.ds(0, _ECH)])
            pltpu.sync_copy(dege.at[pl.ds(b, _ECH)], dcv)

            def _wv(j, _):
                cv = wcv[pl.ds(j * 16, 16)]
                dv = dcv[pl.ds(j * 16, 16)]
                wcv[pl.ds(j * 16, 16)] = dv / jnp.maximum(cv, 1.0)
                return 0
            lax.fori_loop(0, _ECH // 16, _wv, 0)
            pltpu.sync_copy(wcv.at[pl.ds(0, _ECH)], w_out.at[pl.ds(b, _ECH)])
            return 0
        lax.fori_loop(0, _NECH, _wc, 0)
        pltpu.sync_copy(zbuf.at[pl.ds(0, 240)],
                        w_out.at[pl.ds(_E + t * 240, 240)])

    def _slice(s_local, _):
        s = c * 16 + s_local

        def _za(m, _):
            pltpu.sync_copy(zrows, acc_e.at[pl.ds(t * _EZT + m * _ZR, _ZR), :])
            return 0
        lax.fori_loop(0, _EZT // _ZR, _za, 0)
        plsc.subcore_barrier()

        def _pha(k, _):
            r0 = t * (_PPT // 128) + k * _IDXR
            pltpu.sync_copy(vidx32.at[pl.ds(r0, _IDXR), :], vi2)

            def _adj(r, _):
                for l in range(8):
                    va2[r, pl.ds(l * 16, 16)] = vi2[r, pl.ds(l * 16, 16)] + s
                return 0
            lax.fori_loop(0, _IDXR, _adj, 0)

            hs = [pltpu.async_copy(xsl.at[va2.at[j]],
                                   rows.at[pl.ds(j * 128, 128), :], sem1)
                  for j in range(_IDXR)]
            pltpu.sync_copy(eidx.at[pl.ds(r0, _IDXR), :], ei2)
            hs2 = []
            for j in range(_IDXR):
                hs[j].wait()
                hs2.append(pltpu.async_copy(rows.at[pl.ds(j * 128, 128), :],
                                            acc_e.at[ei2.at[j]], sem2,
                                            add=True))
            for h in hs2:
                h.wait()
            return 0
        lax.fori_loop(0, _NCHUNK, _pha, 0)
        plsc.subcore_barrier()

        pltpu.sync_copy(acc_e.at[pl.ds(t * _EZT, _EZT), :],
                        xe_out.at[pl.ds(t * _EZT, _EZT), s, :])
        return 0

    lax.fori_loop(0, 16, _slice, 0)


def _scb_body(xsl, eidx32, vidx, out,
              acc_v, rows, zrows, vi2, ei2, ea2, sem1, sem2):
    c = lax.axis_index("c")
    t = lax.axis_index("s")

    for i in range(_ZR // 16):
        pltpu.sync_copy(xsl.at[pl.ds(_E * _S, 16), :],
                        zrows.at[pl.ds(i * 16, 16), :])

    def _slice(s_local, _):
        s = c * 16 + s_local

        pltpu.sync_copy(zrows, acc_v.at[pl.ds(t * _AVZT, _ZR), :])
        pltpu.sync_copy(zrows, acc_v.at[pl.ds(t * _AVZT + _ZR, _ZR), :])
        plsc.subcore_barrier()

        def _phb(k, _):
            r0 = t * (_PPT // 128) + k * _IDXR
            pltpu.sync_copy(eidx32.at[pl.ds(r0, _IDXR), :], ei2)

            def _adj(r, _):
                for l in range(8):
                    ea2[r, pl.ds(l * 16, 16)] = ei2[r, pl.ds(l * 16, 16)] + s
                return 0
            lax.fori_loop(0, _IDXR, _adj, 0)

            hs = [pltpu.async_copy(xsl.at[ea2.at[j]],
                                   rows.at[pl.ds(j * 128, 128), :], sem1)
                  for j in range(_IDXR)]
            pltpu.sync_copy(vidx.at[pl.ds(r0, _IDXR), :], vi2)
            hs2 = []
            for j in range(_IDXR):
                hs[j].wait()
                hs2.append(pltpu.async_copy(rows.at[pl.ds(j * 128, 128), :],
                                            acc_v.at[vi2.at[j]], sem2,
                                            add=True))
            for h in hs2:
                h.wait()
            return 0
        lax.fori_loop(0, _NCHUNK, _phb, 0)
        plsc.subcore_barrier()

        pltpu.sync_copy(acc_v.at[pl.ds(t * _OV0, _OV0), :],
                        out.at[pl.ds(t * _OV0, _OV0), s, :])

        @pl.when(t == _NS - 1)
        def _tail():
            pltpu.sync_copy(acc_v.at[pl.ds(_NS * _OV0, _OV1 - _OV0), :],
                            out.at[pl.ds(_NS * _OV0, _OV1 - _OV0), s, :])
        plsc.subcore_barrier()
        return 0

    lax.fori_loop(0, 16, _slice, 0)


def _make_sca(compute_w):
    mesh = plsc.VectorSubcoreMesh(core_axis_name="c", subcore_axis_name="s",
                                  num_cores=_NC, num_subcores=_NS)
    out_type = [jax.ShapeDtypeStruct((_EPAD, _S, _W), jnp.float32)]
    scratch = [pltpu.VMEM_SHARED((_EPAD, _W), jnp.float32)]
    if compute_w:
        out_type.append(jax.ShapeDtypeStruct((_EPAD,), jnp.float32))
        scratch.append(pltpu.VMEM_SHARED((_EPAD,), jnp.float32))
    scratch += [
        pltpu.VMEM((_CH, _W), jnp.float32),
        pltpu.VMEM((_ZR, _W), jnp.float32),
        pltpu.VMEM((640,), jnp.float32),
        pltpu.VMEM((128,), jnp.float32),
        pltpu.VMEM((_IDXR, 128), jnp.int32),
        pltpu.VMEM((_IDXR, 128), jnp.int32),
        pltpu.VMEM((_IDXR, 128), jnp.int32),
    ]
    if compute_w:
        scratch += [
            pltpu.VMEM((_ECH,), jnp.float32),
            pltpu.VMEM((_ECH,), jnp.float32),
        ]
    scratch += [pltpu.SemaphoreType.DMA, pltpu.SemaphoreType.DMA]
    return pl.kernel(functools.partial(_sca_body, compute_w),
                     out_type=tuple(out_type), mesh=mesh,
                     scratch_types=scratch,
                     compiler_params=pltpu.CompilerParams(
                         use_tc_tiling_on_sc=False))


def _make_scb():
    mesh = plsc.VectorSubcoreMesh(core_axis_name="c", subcore_axis_name="s",
                                  num_cores=_NC, num_subcores=_NS)
    scratch = [
        pltpu.VMEM_SHARED((_AVROWS, _W), jnp.float32),
        pltpu.VMEM((_CH, _W), jnp.float32),
        pltpu.VMEM((_ZR, _W), jnp.float32),
        pltpu.VMEM((_IDXR, 128), jnp.int32),
        pltpu.VMEM((_IDXR, 128), jnp.int32),
        pltpu.VMEM((_IDXR, 128), jnp.int32),
        pltpu.SemaphoreType.DMA,
        pltpu.SemaphoreType.DMA,
    ]
    return pl.kernel(_scb_body,
                     out_type=(jax.ShapeDtypeStruct((_N, _S, _W),
                                                    jnp.float32),),
                     mesh=mesh, scratch_types=scratch,
                     compiler_params=pltpu.CompilerParams(
                         use_tc_tiling_on_sc=False))


_sca_l1 = _make_sca(True)
_sca_l2 = _make_sca(False)
_scb = _make_scb()


def _scale_body(xe_ref, w_ref, o_ref):
    o_ref[...] = xe_ref[...] * w_ref[...]


_SC_BS = 2048


def _tc_scale(xe, w):
    grid = (_EPAD // _SC_BS,)
    return pl.pallas_call(
        _scale_body,
        grid=grid,
        in_specs=[
            pl.BlockSpec((_SC_BS, _D), lambda i: (i, 0)),
            pl.BlockSpec((_SC_BS, 1), lambda i: (i, 0)),
        ],
        out_specs=pl.BlockSpec((_SC_BS, _D), lambda i: (i, 0)),
        out_shape=jax.ShapeDtypeStruct((_EPAD, _D), jnp.float32),
    )(xe, w)


def _tc_body(x_ref, xv_ref, dv_ref, w_ref, b_ref, o_ref):
    a = x_ref[...] + xv_ref[...] * dv_ref[...]
    y = jnp.dot(a, w_ref[...], preferred_element_type=jnp.float32)
    y = y + b_ref[...]
    nrm = jnp.sqrt(jnp.sum(y * y, axis=1, keepdims=True)) + 1e-12
    o_ref[...] = jnp.maximum(y / nrm, 0.0)


_TC_BS = 1000


def _tc_layer(x, xv, degv2, w, b):
    grid = (_N // _TC_BS,)
    return pl.pallas_call(
        _tc_body,
        grid=grid,
        in_specs=[
            pl.BlockSpec((_TC_BS, _D), lambda i: (i, 0)),
            pl.BlockSpec((_TC_BS, _D), lambda i: (i, 0)),
            pl.BlockSpec((_TC_BS, 1), lambda i: (i, 0)),
            pl.BlockSpec((_D, _D), lambda i: (0, 0)),
            pl.BlockSpec((1, _D), lambda i: (0, 0)),
        ],
        out_specs=pl.BlockSpec((_TC_BS, _D), lambda i: (i, 0)),
        out_shape=jax.ShapeDtypeStruct((_N, _D), jnp.float32),
    )(x, xv, degv2, w, b)


def _flat_layout(x):
    return jnp.pad(x, ((0, 8), (0, 0))).reshape(_XROWS, _W)


def kernel(vertex, edges, degE, degV, user_emb, item_emb, W1, b1, W2, b2):
    x = jnp.concatenate([user_emb, item_emb], axis=0)

    npad = _NNZP - _NNZ
    pad_i = jnp.arange(npad, dtype=jnp.int32)
    vfull = jnp.concatenate([vertex, _N + (pad_i % 8)])
    efull = jnp.concatenate([edges, _E + (pad_i % 2048)])
    vp = vfull.reshape(_NNZP // 128, 128)
    vp32 = (vfull * _S).reshape(_NNZP // 128, 128)
    ep = efull.reshape(_NNZP // 128, 128)
    ep32 = (efull * _S).reshape(_NNZP // 128, 128)
    degv2 = degV.reshape(_N, 1)
    b1r = b1.reshape(1, _D)
    b2r = b2.reshape(1, _D)

    xflat = _flat_layout(x)
    xe1, w = _sca_l1(xflat, vp32, ep, degE)
    w2d = w.reshape(_EPAD, 1)
    xe1_s = _tc_scale(xe1.reshape(_EPAD, _D), w2d)
    (xv1,) = _scb(xe1_s.reshape(_EPAD * _S, _W), ep32, vp)
    x1 = _tc_layer(x, xv1.reshape(_N, _D), degv2, W1, b1r)

    x1flat = _flat_layout(x1)
    (xe2,) = _sca_l2(x1flat, vp32, ep)
    xe2_s = _tc_scale(xe2.reshape(_EPAD, _D), w2d)
    (xv2,) = _scb(xe2_s.reshape(_EPAD * _S, _W), ep32, vp)
    x2 = _tc_layer(x1, xv2.reshape(_N, _D), degv2, W2, b2r)
    return (x2[:_NU], x2[_NU:])
